# Initial kernel scaffold; baseline (speedup 1.0000x reference)
import jax
import jax.numpy as jnp
from jax.experimental import pallas as pl


def kernel(inputs, edge_index0, edge_weight0, edge_index1, edge_weight1, W0, W1):
    raise NotImplementedError("write your pallas kernel here")


# speedup vs baseline: 2.3769x; 2.3769x over previous
import functools

import jax
import jax.numpy as jnp
from jax import lax
from jax.experimental import pallas as pl
from jax.experimental.pallas import tpu as pltpu
from jax.experimental.pallas import tpu_sc as plsc

_B, _N, _D, _E = 4, 10000, 128, 320000
_NS = 16
_NP = 10240
_RPT = _NP // _NS
_EPT = _E // _NS
_K = 128
_NCH = _EPT // _K
_TAIL = _EPT - _NCH * _K
_NB = 1000


def _mm_body(x_ref, w_ref, o_ref):
    o_ref[0, 0] = jnp.dot(x_ref[0], w_ref[0],
                          preferred_element_type=jnp.float32)


def _matmul(x, ws):
    return pl.pallas_call(
        _mm_body,
        grid=(2, _B, _N // _NB),
        in_specs=[
            pl.BlockSpec((1, _NB, _D), lambda s, b, n: (b, n, 0)),
            pl.BlockSpec((1, _D, _D), lambda s, b, n: (s, 0, 0)),
        ],
        out_specs=pl.BlockSpec((1, 1, _NB, _D), lambda s, b, n: (s, b, n, 0)),
        out_shape=jax.ShapeDtypeStruct((2, _B, _N, _D), jnp.float32),
    )(x, ws)


def _sc_body(p_hbm, src_h, dst_h, ew_h, out_hbm,
             rows_v, rows_t, srcv, dstv, ewv, srct, dstt, ewt, accum, sem):
    cid = lax.axis_index("c")
    sid = lax.axis_index("s")
    row0 = sid * _RPT
    ebase = cid * _E + sid * _EPT
    z16 = jnp.zeros((16,), jnp.float32)

    jidx = [jnp.full((16,), j, jnp.int32) for j in range(16)]

    def _scale(rows_ref, ew_ref, k):
        def body(bk, carry):
            ewb16 = ew_ref[pl.ds(bk * 16, 16)]
            for j in range(16):
                ewb = ewb16.at[jidx[j]].get(mode="promise_in_bounds")
                i = bk * 16 + j
                for q in range(_D // 16):
                    sl = pl.ds(q * 16, 16)
                    rows_ref[i, sl] = rows_ref[i, sl] * ewb
            return carry
        lax.fori_loop(0, k // 16, body, 0)

    for b in range(_B):
        def zrow(r, carry):
            for j in range(_D // 16):
                rows_v[r, pl.ds(j * 16, 16)] = z16
            return carry
        lax.fori_loop(0, _K, zrow, 0)
        for off in range(0, _RPT, _K):
            pltpu.sync_copy(rows_v, accum.at[pl.ds(row0 + off, _K)])
        plsc.subcore_barrier()

        def chunk(ch, carry):
            base = ebase + ch * _K
            pltpu.sync_copy(src_h.at[pl.ds(base, _K)], srcv)
            pltpu.sync_copy(dst_h.at[pl.ds(base, _K)], dstv)
            pltpu.sync_copy(ew_h.at[pl.ds(base, _K)], ewv)
            pltpu.async_copy(p_hbm.at[cid, b].at[srcv], rows_v, sem).wait()
            _scale(rows_v, ewv, _K)
            pltpu.sync_copy(rows_v, accum.at[dstv], add=True)
            return carry
        lax.fori_loop(0, _NCH, chunk, 0)
        if _TAIL:
            base = ebase + _NCH * _K
            pltpu.sync_copy(src_h.at[pl.ds(base, _TAIL)], srct)
            pltpu.sync_copy(dst_h.at[pl.ds(base, _TAIL)], dstt)
            pltpu.sync_copy(ew_h.at[pl.ds(base, _TAIL)], ewt)
            pltpu.async_copy(p_hbm.at[cid, b].at[srct], rows_t, sem).wait()
            _scale(rows_t, ewt, _TAIL)
            pltpu.sync_copy(rows_t, accum.at[dstt], add=True)
        plsc.subcore_barrier()

        for off in range(0, _RPT, _K):
            pltpu.sync_copy(accum.at[pl.ds(row0 + off, _K)], rows_v)
            pltpu.sync_copy(rows_v, out_hbm.at[cid, b, pl.ds(row0 + off, _K)])
        plsc.subcore_barrier()


_sc_spmm = functools.partial(
    pl.kernel,
    out_type=jax.ShapeDtypeStruct((2, _B, _NP, _D), jnp.float32),
    mesh=plsc.VectorSubcoreMesh(core_axis_name="c", subcore_axis_name="s"),
    scratch_types=[
        pltpu.VMEM((_K, _D), jnp.float32),
        pltpu.VMEM((_TAIL, _D), jnp.float32),
        pltpu.VMEM((_K,), jnp.int32),
        pltpu.VMEM((_K,), jnp.int32),
        pltpu.VMEM((_K,), jnp.float32),
        pltpu.VMEM((_TAIL,), jnp.int32),
        pltpu.VMEM((_TAIL,), jnp.int32),
        pltpu.VMEM((_TAIL,), jnp.float32),
        pltpu.VMEM_SHARED((_NP, _D), jnp.float32),
        pltpu.SemaphoreType.DMA,
    ],
)(_sc_body)


def kernel(inputs, edge_index0, edge_weight0, edge_index1, edge_weight1,
           W0, W1):
    ws = jnp.stack([W0, W1])
    p = _matmul(inputs, ws)
    src = jnp.concatenate([edge_index0[1], edge_index1[1]])
    dst = jnp.concatenate([edge_index0[0], edge_index1[0]])
    ew = jnp.concatenate([edge_weight0, edge_weight1])
    res = _sc_spmm(p, src, dst, ew)
    return jnp.concatenate([res[0, :, :_N], res[1, :, :_N]], axis=-1)
